# baseline (speedup 1.0000x reference)
import functools

import jax
import jax.numpy as jnp
from jax import lax
from jax.experimental import pallas as pl
from jax.experimental.pallas import tpu as pltpu
from jax.experimental.pallas import tpu_sc as plsc

N_NODES = 10000
N_EDGES = 320000
N_FEAT = 128
NEG_SLOPE = 0.05

NW = 32
CH = 128
NCH = 80
EPT = NCH * CH
E_PAD = NW * EPT
NCH_LAST = (N_EDGES - (NW - 1) * EPT) // CH
NPAD = 10240
SL = NPAD // 16
FH = N_FEAT // 2

_mesh = plsc.VectorSubcoreMesh(core_axis_name="c", subcore_axis_name="s")
_sc_params = pltpu.CompilerParams(needs_layout_passes=False,
                                  use_tc_tiling_on_sc=False)


def _k1_body(x_ref, w_ref, a_ref, b_ref, z0_ref, z1_ref, s_ref):
    z = lax.dot_general(x_ref[...], w_ref[...],
                        (((1,), (1,)), ((), ())),
                        preferred_element_type=jnp.float32)

    def _pack_half(zh):
        n = zh.shape[0]
        blocks = []
        for m in range(FH // 32):
            u = zh[:, m * 32:m * 32 + 16]
            v = zh[:, m * 32 + 16:m * 32 + 32]
            blocks.append(jnp.stack([u, v], axis=-1).reshape(n, 32))
        return jnp.concatenate(blocks, axis=1).astype(jnp.bfloat16)

    z0_ref[...] = _pack_half(z[:, :FH])
    z1_ref[...] = _pack_half(z[:, FH:])
    s_ref[...] = lax.dot_general(a_ref[...], z,
                                 (((1,), (1,)), ((), ())),
                                 preferred_element_type=jnp.float32) + b_ref[...]


def _k1(x_pad, w, a_pair, bias):
    nb = NPAD // 1024
    return pl.pallas_call(
        _k1_body,
        grid=(nb,),
        in_specs=[pl.BlockSpec((1024, N_FEAT), lambda i: (i, 0)),
                  pl.BlockSpec((N_FEAT, N_FEAT), lambda i: (0, 0)),
                  pl.BlockSpec((2, N_FEAT), lambda i: (0, 0)),
                  pl.BlockSpec((2, 1), lambda i: (0, 0))],
        out_specs=(pl.BlockSpec((1024, FH), lambda i: (i, 0)),
                   pl.BlockSpec((1024, FH), lambda i: (i, 0)),
                   pl.BlockSpec((2, 1024), lambda i: (0, i))),
        out_shape=(jax.ShapeDtypeStruct((NPAD, FH), jnp.bfloat16),
                   jax.ShapeDtypeStruct((NPAD, FH), jnp.bfloat16),
                   jax.ShapeDtypeStruct((2, NPAD), jnp.float32)),
    )(x_pad, w, a_pair, bias)


@functools.partial(
    pl.kernel, mesh=_mesh, compiler_params=_sc_params,
    out_type=[jax.ShapeDtypeStruct((NW, NCH, CH), jnp.float32),
              jax.ShapeDtypeStruct((2, NPAD), jnp.float32)],
    scratch_types=[
        pltpu.VMEM((NCH, CH), jnp.int32),
        pltpu.VMEM((NCH, CH), jnp.int32),
        pltpu.VMEM((NPAD,), jnp.float32),
        pltpu.VMEM((NPAD,), jnp.float32),
        pltpu.VMEM((NCH, CH), jnp.float32),
        pltpu.VMEM((SL,), jnp.float32),
        pltpu.VMEM_SHARED((NPAD,), jnp.float32),
        pltpu.SemaphoreType.DMA,
    ],
)
def _k2(ei_hbm, s_hbm, h_hbm, hsum_hbm,
        src_v, dst_v, s1_v, s2_v, h_v, zb_v, hsum_sh, sem):
    cid = lax.axis_index("c")
    sid = lax.axis_index("s")
    wid = sid * 2 + cid
    nch = jnp.where(wid == NW - 1, NCH_LAST, NCH)
    pltpu.sync_copy(ei_hbm.at[0, wid], src_v)
    pltpu.sync_copy(ei_hbm.at[1, wid], dst_v)
    pltpu.sync_copy(s_hbm.at[0], s1_v)
    pltpu.sync_copy(s_hbm.at[1], s2_v)

    def _zb(i, c):
        zb_v[pl.ds(i * 16, 16)] = jnp.zeros((16,), jnp.float32)
        return c
    lax.fori_loop(0, SL // 16, _zb, 0)
    pltpu.sync_copy(zb_v, hsum_sh.at[pl.ds(sid * SL, SL)])
    plsc.subcore_barrier()

    def _chunk(j, c):
        for g in range(CH // 16):
            si = src_v[j, pl.ds(g * 16, 16)]
            di = dst_v[j, pl.ds(g * 16, 16)]
            v = plsc.load_gather(s1_v, [si]) + plsc.load_gather(s2_v, [di])
            h_v[j, pl.ds(g * 16, 16)] = jnp.exp(
                jnp.where(v >= 0, v, v * NEG_SLOPE))
        return c
    lax.fori_loop(0, nch, _chunk, 0)
    pltpu.sync_copy(h_v, h_hbm.at[wid])

    def _scat(j, c):
        pltpu.async_copy(h_v.at[j], hsum_sh.at[src_v.at[j]], sem, add=True)
        return c
    lax.fori_loop(0, nch, _scat, 0)

    def _drain(j, c):
        pltpu.make_async_copy(h_v.at[0], hsum_sh.at[src_v.at[0]], sem).wait()
        return c
    lax.fori_loop(0, nch, _drain, 0)
    plsc.subcore_barrier()
    pltpu.sync_copy(hsum_sh.at[pl.ds(sid * SL, SL)],
                    hsum_hbm.at[cid, pl.ds(sid * SL, SL)])


@functools.partial(
    pl.kernel, mesh=_mesh, compiler_params=_sc_params,
    out_type=[jax.ShapeDtypeStruct((NW, NCH, CH), jnp.float32),
              jax.ShapeDtypeStruct((2, 2, NPAD, FH), jnp.float32)],
    scratch_types=[
        pltpu.VMEM((NCH, CH), jnp.int32),
        pltpu.VMEM((NCH, CH), jnp.int32),
        pltpu.VMEM((NCH, CH), jnp.float32),
        pltpu.VMEM((NPAD,), jnp.float32),
        pltpu.VMEM((NPAD,), jnp.float32),
        pltpu.VMEM((CH, FH), jnp.bfloat16),
        pltpu.VMEM((CH, FH), jnp.bfloat16),
        pltpu.VMEM((CH, FH), jnp.float32),
        pltpu.VMEM((CH, FH), jnp.float32),
        pltpu.VMEM_SHARED((NPAD, FH), jnp.float32),
        pltpu.SemaphoreType.DMA,
        pltpu.SemaphoreType.DMA,
        pltpu.SemaphoreType.DMA,
        pltpu.SemaphoreType.DMA,
    ],
)
def _k3(ei_hbm, h_hbm, hsp_hbm, z0_hbm, z1_hbm, al_hbm, outp_hbm,
        src_v, dst_v, h_v, hs_v, hs2_v, rows0_v, rows1_v,
        sb0_v, sb1_v, acc_sh, semg0, semg1, sems0, sems1):
    cid = lax.axis_index("c")
    sid = lax.axis_index("s")
    wid = sid * 2 + cid
    nch = jnp.where(wid == NW - 1, NCH_LAST, NCH)
    pltpu.sync_copy(ei_hbm.at[0, wid], src_v)
    pltpu.sync_copy(ei_hbm.at[1, wid], dst_v)
    pltpu.sync_copy(h_hbm.at[wid], h_v)
    pltpu.sync_copy(hsp_hbm.at[0], hs_v)
    pltpu.sync_copy(hsp_hbm.at[1], hs2_v)

    def _add(i, c):
        hs_v[pl.ds(i * 16, 16)] = (hs_v[pl.ds(i * 16, 16)]
                                   + hs2_v[pl.ds(i * 16, 16)])
        return c
    lax.fori_loop(0, NPAD // 16, _add, 0)

    def _alpha(j, c):
        for g in range(CH // 16):
            si = src_v[j, pl.ds(g * 16, 16)]
            hv = h_v[j, pl.ds(g * 16, 16)]
            h_v[j, pl.ds(g * 16, 16)] = hv / plsc.load_gather(hs_v, [si])
        return c
    lax.fori_loop(0, nch, _alpha, 0)
    pltpu.sync_copy(h_v, al_hbm.at[wid])

    def _zr(e, c):
        for k in range(FH // 16):
            sb0_v[e, pl.ds(k * 16, 16)] = jnp.zeros((16,), jnp.float32)
        return c
    lax.fori_loop(0, CH, _zr, 0)
    for kk in range(SL // CH):
        pltpu.sync_copy(sb0_v, acc_sh.at[pl.ds(sid * SL + kk * CH, CH)])

    def _scale(rows, sb, j):
        @plsc.parallel_loop(0, CH, unroll=4)
        def _edge(e):
            jf = jnp.full((16,), j, jnp.int32)
            ef = jnp.full((16,), e, jnp.int32)
            asp = plsc.load_gather(h_v, [jf, ef])
            for m in range(FH // 32):
                v = rows[e, pl.ds(m * 32, 32)]
                lo, hi = plsc.unpack(v, format=plsc.PackFormat.INTERLEAVED)
                sb[e, pl.ds(m * 32, 16)] = lo * asp
                sb[e, pl.ds(m * 32 + 16, 16)] = hi * asp

    for half, z_hbm in enumerate((z0_hbm, z1_hbm)):
        plsc.subcore_barrier()
        pltpu.async_copy(z_hbm.at[dst_v.at[0]], rows0_v, semg0)
        pltpu.async_copy(z_hbm.at[dst_v.at[1]], rows1_v, semg1)

        def _pair(i, c):
            j0 = 2 * i
            pltpu.make_async_copy(z_hbm.at[dst_v.at[j0]], rows0_v,
                                  semg0).wait()

            @pl.when(i > 0)
            def _():
                pltpu.make_async_copy(sb0_v, acc_sh.at[src_v.at[j0]],
                                      sems0).wait()
            _scale(rows0_v, sb0_v, j0)

            @pl.when(j0 + 2 < nch)
            def _():
                pltpu.async_copy(z_hbm.at[dst_v.at[j0 + 2]], rows0_v, semg0)
            pltpu.async_copy(sb0_v, acc_sh.at[src_v.at[j0]], sems0,
                             add=True)

            j1 = 2 * i + 1
            pltpu.make_async_copy(z_hbm.at[dst_v.at[j1]], rows1_v,
                                  semg1).wait()

            @pl.when(i > 0)
            def _():
                pltpu.make_async_copy(sb1_v, acc_sh.at[src_v.at[j1]],
                                      sems1).wait()
            _scale(rows1_v, sb1_v, j1)

            @pl.when(j1 + 2 < nch)
            def _():
                pltpu.async_copy(z_hbm.at[dst_v.at[j1 + 2]], rows1_v, semg1)
            pltpu.async_copy(sb1_v, acc_sh.at[src_v.at[j1]], sems1,
                             add=True)
            return c
        lax.fori_loop(0, nch // 2, _pair, 0)
        pltpu.make_async_copy(sb0_v, acc_sh.at[src_v.at[0]], sems0).wait()
        pltpu.make_async_copy(sb1_v, acc_sh.at[src_v.at[0]], sems1).wait()

        plsc.subcore_barrier()
        pltpu.sync_copy(acc_sh.at[pl.ds(sid * SL, SL)],
                        outp_hbm.at[cid, half, pl.ds(sid * SL, SL)])
        if half == 0:
            lax.fori_loop(0, CH, _zr, 0)
            for kk in range(SL // CH):
                pltpu.sync_copy(
                    sb0_v, acc_sh.at[pl.ds(sid * SL + kk * CH, CH)])


def _k4_body(p_ref, o_ref):
    o_ref[:, :FH] = p_ref[0, 0] + p_ref[1, 0]
    o_ref[:, FH:] = p_ref[0, 1] + p_ref[1, 1]


def _k4(outp):
    return pl.pallas_call(
        _k4_body,
        grid=(10,),
        in_specs=[pl.BlockSpec((2, 2, 1000, FH), lambda i: (0, 0, i, 0))],
        out_specs=pl.BlockSpec((1000, N_FEAT), lambda i: (i, 0)),
        out_shape=jax.ShapeDtypeStruct((N_NODES, N_FEAT), jnp.float32),
    )(outp)


def kernel(x, edge_index, W_fc, a_w, a_b):
    ei = jnp.pad(edge_index.astype(jnp.int32),
                 ((0, 0), (0, E_PAD - N_EDGES))).reshape(2, NW, NCH, CH)
    a_pair = a_w.reshape(2, N_FEAT)
    bias = jnp.stack([a_b, jnp.zeros_like(a_b)]).reshape(2, 1)
    x_pad = jnp.pad(x, ((0, NPAD - N_NODES), (0, 0)))
    z0, z1, s_pair = _k1(x_pad, W_fc, a_pair, bias)
    h, hsum_p = _k2(ei, s_pair)
    alpha, outp = _k3(ei, h, hsum_p, z0, z1)
    out = _k4(outp)
    return out, alpha.reshape(E_PAD)[:N_EDGES]


# speedup vs baseline: 1.9500x; 1.9500x over previous
import functools

import jax
import jax.numpy as jnp
import numpy as np
from jax import lax
from jax.experimental import pallas as pl
from jax.experimental.pallas import tpu as pltpu
from jax.experimental.pallas import tpu_sc as plsc

N_NODES = 10000
N_EDGES = 320000
N_FEAT = 128
NEG_SLOPE = 0.05

NW = 32
CH = 128
NCH = 80
EPT = NCH * CH
E_PAD = NW * EPT
NCH_LAST = (N_EDGES - (NW - 1) * EPT) // CH
NPAD = 10240
SL = NPAD // 16
FH = N_FEAT // 2

_PERM = np.empty((N_FEAT,), np.int32)
for _m in range(N_FEAT // 32):
    for _t in range(16):
        _PERM[32 * _m + 2 * _t] = 32 * _m + _t
        _PERM[32 * _m + 2 * _t + 1] = 32 * _m + 16 + _t

_mesh = plsc.VectorSubcoreMesh(core_axis_name="c", subcore_axis_name="s")
_sc_params = pltpu.CompilerParams(needs_layout_passes=False,
                                  use_tc_tiling_on_sc=False)


def _k1_body(x_ref, w_ref, a_ref, b_ref, z0_ref, z1_ref, s_ref):
    z = lax.dot_general(x_ref[...], w_ref[...],
                        (((1,), (1,)), ((), ())),
                        preferred_element_type=jnp.float32)

    z0_ref[...] = z[:, :FH].astype(jnp.bfloat16)
    z1_ref[...] = z[:, FH:].astype(jnp.bfloat16)
    s_ref[...] = lax.dot_general(a_ref[...], z,
                                 (((1,), (1,)), ((), ())),
                                 preferred_element_type=jnp.float32) + b_ref[...]


def _k1(x_pad, w, a_pair, bias):
    nb = NPAD // 1024
    return pl.pallas_call(
        _k1_body,
        grid=(nb,),
        in_specs=[pl.BlockSpec((1024, N_FEAT), lambda i: (i, 0)),
                  pl.BlockSpec((N_FEAT, N_FEAT), lambda i: (0, 0)),
                  pl.BlockSpec((2, N_FEAT), lambda i: (0, 0)),
                  pl.BlockSpec((2, 1), lambda i: (0, 0))],
        out_specs=(pl.BlockSpec((1024, FH), lambda i: (i, 0)),
                   pl.BlockSpec((1024, FH), lambda i: (i, 0)),
                   pl.BlockSpec((2, 1024), lambda i: (0, i))),
        out_shape=(jax.ShapeDtypeStruct((NPAD, FH), jnp.bfloat16),
                   jax.ShapeDtypeStruct((NPAD, FH), jnp.bfloat16),
                   jax.ShapeDtypeStruct((2, NPAD), jnp.float32)),
    )(x_pad, w, a_pair, bias)


@functools.partial(
    pl.kernel, mesh=_mesh, compiler_params=_sc_params,
    out_type=[jax.ShapeDtypeStruct((NW, NCH, CH), jnp.float32),
              jax.ShapeDtypeStruct((2, NPAD), jnp.float32)],
    scratch_types=[
        pltpu.VMEM((NCH, CH), jnp.int32),
        pltpu.VMEM((NCH, CH), jnp.int32),
        pltpu.VMEM((NPAD,), jnp.float32),
        pltpu.VMEM((NPAD,), jnp.float32),
        pltpu.VMEM((NCH, CH), jnp.float32),
        pltpu.VMEM((SL,), jnp.float32),
        pltpu.VMEM_SHARED((NPAD,), jnp.float32),
        pltpu.SemaphoreType.DMA,
    ],
)
def _k2(ei_hbm, s_hbm, h_hbm, hsum_hbm,
        src_v, dst_v, s1_v, s2_v, h_v, zb_v, hsum_sh, sem):
    cid = lax.axis_index("c")
    sid = lax.axis_index("s")
    wid = sid * 2 + cid
    nch = jnp.where(wid == NW - 1, NCH_LAST, NCH)
    pltpu.sync_copy(ei_hbm.at[0, wid], src_v)
    pltpu.sync_copy(ei_hbm.at[1, wid], dst_v)
    pltpu.sync_copy(s_hbm.at[0], s1_v)
    pltpu.sync_copy(s_hbm.at[1], s2_v)

    def _zb(i, c):
        zb_v[pl.ds(i * 16, 16)] = jnp.zeros((16,), jnp.float32)
        return c
    lax.fori_loop(0, SL // 16, _zb, 0)
    pltpu.sync_copy(zb_v, hsum_sh.at[pl.ds(sid * SL, SL)])
    plsc.subcore_barrier()

    def _chunk(j, c):
        for g in range(CH // 16):
            si = src_v[j, pl.ds(g * 16, 16)]
            di = dst_v[j, pl.ds(g * 16, 16)]
            v = plsc.load_gather(s1_v, [si]) + plsc.load_gather(s2_v, [di])
            h_v[j, pl.ds(g * 16, 16)] = jnp.exp(
                jnp.where(v >= 0, v, v * NEG_SLOPE))
        return c
    lax.fori_loop(0, nch, _chunk, 0)
    pltpu.sync_copy(h_v, h_hbm.at[wid])

    def _scat(j, c):
        pltpu.async_copy(h_v.at[j], hsum_sh.at[src_v.at[j]], sem, add=True)
        return c
    lax.fori_loop(0, nch, _scat, 0)

    def _drain(j, c):
        pltpu.make_async_copy(h_v.at[0], hsum_sh.at[src_v.at[0]], sem).wait()
        return c
    lax.fori_loop(0, nch, _drain, 0)
    plsc.subcore_barrier()
    pltpu.sync_copy(hsum_sh.at[pl.ds(sid * SL, SL)],
                    hsum_hbm.at[cid, pl.ds(sid * SL, SL)])


@functools.partial(
    pl.kernel, mesh=_mesh, compiler_params=_sc_params,
    out_type=[jax.ShapeDtypeStruct((NW, NCH, CH), jnp.float32),
              jax.ShapeDtypeStruct((2, 2, NPAD, FH), jnp.float32)],
    scratch_types=[
        pltpu.VMEM((NCH, CH), jnp.int32),
        pltpu.VMEM((NCH, CH), jnp.int32),
        pltpu.VMEM((NCH, CH), jnp.float32),
        pltpu.VMEM((NPAD,), jnp.float32),
        pltpu.VMEM((NPAD,), jnp.float32),
        pltpu.VMEM((CH, FH), jnp.bfloat16),
        pltpu.VMEM((CH, FH), jnp.bfloat16),
        pltpu.VMEM((CH, FH), jnp.float32),
        pltpu.VMEM((CH, FH), jnp.float32),
        pltpu.VMEM_SHARED((NPAD, FH), jnp.float32),
        pltpu.SemaphoreType.DMA,
        pltpu.SemaphoreType.DMA,
        pltpu.SemaphoreType.DMA,
        pltpu.SemaphoreType.DMA,
    ],
)
def _k3(ei_hbm, h_hbm, hsp_hbm, z0_hbm, z1_hbm, al_hbm, outp_hbm,
        src_v, dst_v, h_v, hs_v, hs2_v, rows0_v, rows1_v,
        sb0_v, sb1_v, acc_sh, semg0, semg1, sems0, sems1):
    cid = lax.axis_index("c")
    sid = lax.axis_index("s")
    wid = sid * 2 + cid
    nch = jnp.where(wid == NW - 1, NCH_LAST, NCH)
    pltpu.sync_copy(ei_hbm.at[0, wid], src_v)
    pltpu.sync_copy(ei_hbm.at[1, wid], dst_v)
    pltpu.sync_copy(h_hbm.at[wid], h_v)
    pltpu.sync_copy(hsp_hbm.at[0], hs_v)
    pltpu.sync_copy(hsp_hbm.at[1], hs2_v)

    def _add(i, c):
        hs_v[pl.ds(i * 16, 16)] = (hs_v[pl.ds(i * 16, 16)]
                                   + hs2_v[pl.ds(i * 16, 16)])
        return c
    lax.fori_loop(0, NPAD // 16, _add, 0)

    def _alpha(j, c):
        for g in range(CH // 16):
            si = src_v[j, pl.ds(g * 16, 16)]
            hv = h_v[j, pl.ds(g * 16, 16)]
            h_v[j, pl.ds(g * 16, 16)] = hv / plsc.load_gather(hs_v, [si])
        return c
    lax.fori_loop(0, nch, _alpha, 0)
    pltpu.sync_copy(h_v, al_hbm.at[wid])

    def _zr(e, c):
        for k in range(FH // 16):
            sb0_v[e, pl.ds(k * 16, 16)] = jnp.zeros((16,), jnp.float32)
        return c
    lax.fori_loop(0, CH, _zr, 0)
    for kk in range(SL // CH):
        pltpu.sync_copy(sb0_v, acc_sh.at[pl.ds(sid * SL + kk * CH, CH)])

    def _scale(rows, sb, j):
        @plsc.parallel_loop(0, CH, unroll=4)
        def _edge(e):
            jf = jnp.full((16,), j, jnp.int32)
            ef = jnp.full((16,), e, jnp.int32)
            asp = plsc.load_gather(h_v, [jf, ef])
            for m in range(FH // 32):
                v = rows[e, pl.ds(m * 32, 32)]
                lo, hi = plsc.unpack(v, format=plsc.PackFormat.INTERLEAVED)
                sb[e, pl.ds(m * 32, 16)] = lo * asp
                sb[e, pl.ds(m * 32 + 16, 16)] = hi * asp

    for half, z_hbm in enumerate((z0_hbm, z1_hbm)):
        plsc.subcore_barrier()
        pltpu.async_copy(z_hbm.at[dst_v.at[0]], rows0_v, semg0)
        pltpu.async_copy(z_hbm.at[dst_v.at[1]], rows1_v, semg1)

        def _pair(i, c):
            j0 = 2 * i
            pltpu.make_async_copy(z_hbm.at[dst_v.at[j0]], rows0_v,
                                  semg0).wait()

            @pl.when(i > 0)
            def _():
                pltpu.make_async_copy(sb0_v, acc_sh.at[src_v.at[j0]],
                                      sems0).wait()
            _scale(rows0_v, sb0_v, j0)

            @pl.when(j0 + 2 < nch)
            def _():
                pltpu.async_copy(z_hbm.at[dst_v.at[j0 + 2]], rows0_v, semg0)
            pltpu.async_copy(sb0_v, acc_sh.at[src_v.at[j0]], sems0,
                             add=True)

            j1 = 2 * i + 1
            pltpu.make_async_copy(z_hbm.at[dst_v.at[j1]], rows1_v,
                                  semg1).wait()

            @pl.when(i > 0)
            def _():
                pltpu.make_async_copy(sb1_v, acc_sh.at[src_v.at[j1]],
                                      sems1).wait()
            _scale(rows1_v, sb1_v, j1)

            @pl.when(j1 + 2 < nch)
            def _():
                pltpu.async_copy(z_hbm.at[dst_v.at[j1 + 2]], rows1_v, semg1)
            pltpu.async_copy(sb1_v, acc_sh.at[src_v.at[j1]], sems1,
                             add=True)
            return c
        lax.fori_loop(0, nch // 2, _pair, 0)
        pltpu.make_async_copy(sb0_v, acc_sh.at[src_v.at[0]], sems0).wait()
        pltpu.make_async_copy(sb1_v, acc_sh.at[src_v.at[0]], sems1).wait()

        plsc.subcore_barrier()
        pltpu.sync_copy(acc_sh.at[pl.ds(sid * SL, SL)],
                        outp_hbm.at[cid, half, pl.ds(sid * SL, SL)])
        if half == 0:
            lax.fori_loop(0, CH, _zr, 0)
            for kk in range(SL // CH):
                pltpu.sync_copy(
                    sb0_v, acc_sh.at[pl.ds(sid * SL + kk * CH, CH)])


def _k4_body(p_ref, o_ref):
    o_ref[:, :FH] = p_ref[0, 0] + p_ref[1, 0]
    o_ref[:, FH:] = p_ref[0, 1] + p_ref[1, 1]


def _k4(outp):
    return pl.pallas_call(
        _k4_body,
        grid=(10,),
        in_specs=[pl.BlockSpec((2, 2, 1000, FH), lambda i: (0, 0, i, 0))],
        out_specs=pl.BlockSpec((1000, N_FEAT), lambda i: (i, 0)),
        out_shape=jax.ShapeDtypeStruct((N_NODES, N_FEAT), jnp.float32),
    )(outp)


def kernel(x, edge_index, W_fc, a_w, a_b):
    ei = jnp.pad(edge_index.astype(jnp.int32),
                 ((0, 0), (0, E_PAD - N_EDGES))).reshape(2, NW, NCH, CH)
    perm = jnp.asarray(_PERM)
    w_p = W_fc[perm, :]
    a_pair = a_w.reshape(2, N_FEAT)[:, perm]
    bias = jnp.stack([a_b, jnp.zeros_like(a_b)]).reshape(2, 1)
    x_pad = jnp.pad(x, ((0, NPAD - N_NODES), (0, 0)))
    z0, z1, s_pair = _k1(x_pad, w_p, a_pair, bias)
    h, hsum_p = _k2(ei, s_pair)
    alpha, outp = _k3(ei, h, hsum_p, z0, z1)
    out = _k4(outp)
    return out, alpha.reshape(E_PAD)[:N_EDGES]


# speedup vs baseline: 1.9800x; 1.0154x over previous
import functools

import jax
import jax.numpy as jnp
import numpy as np
from jax import lax
from jax.experimental import pallas as pl
from jax.experimental.pallas import tpu as pltpu
from jax.experimental.pallas import tpu_sc as plsc

N_NODES = 10000
N_EDGES = 320000
N_FEAT = 128
NEG_SLOPE = 0.05

NW = 32
CH = 128
NCH = 80
EPT = NCH * CH
E_PAD = NW * EPT
NCH_LAST = (N_EDGES - (NW - 1) * EPT) // CH
NPAD = 10240
SL = NPAD // 16
FH = N_FEAT // 2

_PERM = np.empty((N_FEAT,), np.int32)
for _m in range(N_FEAT // 32):
    for _t in range(16):
        _PERM[32 * _m + 2 * _t] = 32 * _m + _t
        _PERM[32 * _m + 2 * _t + 1] = 32 * _m + 16 + _t

_mesh = plsc.VectorSubcoreMesh(core_axis_name="c", subcore_axis_name="s")
_sc_params = pltpu.CompilerParams(needs_layout_passes=False,
                                  use_tc_tiling_on_sc=False)


def _k1_body(x_ref, w_ref, a_ref, b_ref, z0_ref, z1_ref, s_ref):
    z = lax.dot_general(x_ref[...], w_ref[...],
                        (((1,), (1,)), ((), ())),
                        preferred_element_type=jnp.float32)

    z0_ref[...] = z[:, :FH].astype(jnp.bfloat16)
    z1_ref[...] = z[:, FH:].astype(jnp.bfloat16)
    s_ref[...] = lax.dot_general(a_ref[...], z,
                                 (((1,), (1,)), ((), ())),
                                 preferred_element_type=jnp.float32) + b_ref[...]


def _k1(x_pad, w, a_pair, bias):
    nb = NPAD // 1024
    return pl.pallas_call(
        _k1_body,
        grid=(nb,),
        in_specs=[pl.BlockSpec((1024, N_FEAT), lambda i: (i, 0)),
                  pl.BlockSpec((N_FEAT, N_FEAT), lambda i: (0, 0)),
                  pl.BlockSpec((2, N_FEAT), lambda i: (0, 0)),
                  pl.BlockSpec((2, 1), lambda i: (0, 0))],
        out_specs=(pl.BlockSpec((1024, FH), lambda i: (i, 0)),
                   pl.BlockSpec((1024, FH), lambda i: (i, 0)),
                   pl.BlockSpec((2, 1024), lambda i: (0, i))),
        out_shape=(jax.ShapeDtypeStruct((NPAD, FH), jnp.bfloat16),
                   jax.ShapeDtypeStruct((NPAD, FH), jnp.bfloat16),
                   jax.ShapeDtypeStruct((2, NPAD), jnp.float32)),
    )(x_pad, w, a_pair, bias)


@functools.partial(
    pl.kernel, mesh=_mesh, compiler_params=_sc_params,
    out_type=[jax.ShapeDtypeStruct((NW, NCH, CH), jnp.float32),
              jax.ShapeDtypeStruct((2, NPAD), jnp.float32)],
    scratch_types=[
        pltpu.VMEM((NCH, CH), jnp.int32),
        pltpu.VMEM((NCH, CH), jnp.int32),
        pltpu.VMEM((NPAD,), jnp.float32),
        pltpu.VMEM((NPAD,), jnp.float32),
        pltpu.VMEM((NCH, CH), jnp.float32),
        pltpu.VMEM((SL,), jnp.float32),
        pltpu.VMEM_SHARED((NPAD,), jnp.float32),
        pltpu.SemaphoreType.DMA,
    ],
)
def _k2(ei_hbm, s_hbm, h_hbm, hsum_hbm,
        src_v, dst_v, s1_v, s2_v, h_v, zb_v, hsum_sh, sem):
    cid = lax.axis_index("c")
    sid = lax.axis_index("s")
    wid = sid * 2 + cid
    nch = jnp.where(wid == NW - 1, NCH_LAST, NCH)
    pltpu.async_copy(ei_hbm.at[0, wid], src_v, sem)
    pltpu.async_copy(ei_hbm.at[1, wid], dst_v, sem)
    pltpu.async_copy(s_hbm.at[0], s1_v, sem)
    pltpu.async_copy(s_hbm.at[1], s2_v, sem)
    pltpu.make_async_copy(ei_hbm.at[0, wid], src_v, sem).wait()
    pltpu.make_async_copy(ei_hbm.at[1, wid], dst_v, sem).wait()
    pltpu.make_async_copy(s_hbm.at[0], s1_v, sem).wait()
    pltpu.make_async_copy(s_hbm.at[1], s2_v, sem).wait()

    def _zb(i, c):
        zb_v[pl.ds(i * 16, 16)] = jnp.zeros((16,), jnp.float32)
        return c
    lax.fori_loop(0, SL // 16, _zb, 0)
    pltpu.sync_copy(zb_v, hsum_sh.at[pl.ds(sid * SL, SL)])
    plsc.subcore_barrier()

    def _chunk(j, c):
        for g in range(CH // 16):
            si = src_v[j, pl.ds(g * 16, 16)]
            di = dst_v[j, pl.ds(g * 16, 16)]
            v = plsc.load_gather(s1_v, [si]) + plsc.load_gather(s2_v, [di])
            h_v[j, pl.ds(g * 16, 16)] = jnp.exp(
                jnp.where(v >= 0, v, v * NEG_SLOPE))
        return c
    lax.fori_loop(0, nch, _chunk, 0)
    pltpu.sync_copy(h_v, h_hbm.at[wid])

    def _scat(j, c):
        pltpu.async_copy(h_v.at[j], hsum_sh.at[src_v.at[j]], sem, add=True)
        return c
    lax.fori_loop(0, nch, _scat, 0)

    def _drain(j, c):
        pltpu.make_async_copy(h_v.at[0], hsum_sh.at[src_v.at[0]], sem).wait()
        return c
    lax.fori_loop(0, nch, _drain, 0)
    plsc.subcore_barrier()
    pltpu.sync_copy(hsum_sh.at[pl.ds(sid * SL, SL)],
                    hsum_hbm.at[cid, pl.ds(sid * SL, SL)])


@functools.partial(
    pl.kernel, mesh=_mesh, compiler_params=_sc_params,
    out_type=[jax.ShapeDtypeStruct((NW, NCH, CH), jnp.float32),
              jax.ShapeDtypeStruct((2, 2, NPAD, FH), jnp.float32)],
    scratch_types=[
        pltpu.VMEM((NCH, CH), jnp.int32),
        pltpu.VMEM((NCH, CH), jnp.int32),
        pltpu.VMEM((NCH, CH), jnp.float32),
        pltpu.VMEM((NPAD,), jnp.float32),
        pltpu.VMEM((NPAD,), jnp.float32),
        pltpu.VMEM((CH, FH), jnp.bfloat16),
        pltpu.VMEM((CH, FH), jnp.bfloat16),
        pltpu.VMEM((CH, FH), jnp.float32),
        pltpu.VMEM((CH, FH), jnp.float32),
        pltpu.VMEM_SHARED((NPAD, FH), jnp.float32),
        pltpu.SemaphoreType.DMA,
        pltpu.SemaphoreType.DMA,
        pltpu.SemaphoreType.DMA,
        pltpu.SemaphoreType.DMA,
    ],
)
def _k3(ei_hbm, h_hbm, hsp_hbm, z0_hbm, z1_hbm, al_hbm, outp_hbm,
        src_v, dst_v, h_v, hs_v, hs2_v, rows0_v, rows1_v,
        sb0_v, sb1_v, acc_sh, semg0, semg1, sems0, sems1):
    cid = lax.axis_index("c")
    sid = lax.axis_index("s")
    wid = sid * 2 + cid
    nch = jnp.where(wid == NW - 1, NCH_LAST, NCH)
    pltpu.async_copy(ei_hbm.at[0, wid], src_v, semg0)
    pltpu.async_copy(ei_hbm.at[1, wid], dst_v, semg0)
    pltpu.async_copy(h_hbm.at[wid], h_v, semg0)
    pltpu.async_copy(hsp_hbm.at[0], hs_v, semg0)
    pltpu.async_copy(hsp_hbm.at[1], hs2_v, semg0)
    pltpu.make_async_copy(ei_hbm.at[0, wid], src_v, semg0).wait()
    pltpu.make_async_copy(ei_hbm.at[1, wid], dst_v, semg0).wait()
    pltpu.make_async_copy(h_hbm.at[wid], h_v, semg0).wait()
    pltpu.make_async_copy(hsp_hbm.at[0], hs_v, semg0).wait()
    pltpu.make_async_copy(hsp_hbm.at[1], hs2_v, semg0).wait()

    def _add(i, c):
        hs_v[pl.ds(i * 16, 16)] = (hs_v[pl.ds(i * 16, 16)]
                                   + hs2_v[pl.ds(i * 16, 16)])
        return c
    lax.fori_loop(0, NPAD // 16, _add, 0)

    def _alpha(j, c):
        for g in range(CH // 16):
            si = src_v[j, pl.ds(g * 16, 16)]
            hv = h_v[j, pl.ds(g * 16, 16)]
            h_v[j, pl.ds(g * 16, 16)] = hv / plsc.load_gather(hs_v, [si])
        return c
    lax.fori_loop(0, nch, _alpha, 0)
    pltpu.sync_copy(h_v, al_hbm.at[wid])

    def _zr(e, c):
        for k in range(FH // 16):
            sb0_v[e, pl.ds(k * 16, 16)] = jnp.zeros((16,), jnp.float32)
        return c
    lax.fori_loop(0, CH, _zr, 0)
    for kk in range(SL // CH):
        pltpu.sync_copy(sb0_v, acc_sh.at[pl.ds(sid * SL + kk * CH, CH)])

    def _scale(rows, sb, j):
        @plsc.parallel_loop(0, CH, unroll=4)
        def _edge(e):
            jf = jnp.full((16,), j, jnp.int32)
            ef = jnp.full((16,), e, jnp.int32)
            asp = plsc.load_gather(h_v, [jf, ef])
            for m in range(FH // 32):
                v = rows[e, pl.ds(m * 32, 32)]
                lo, hi = plsc.unpack(v, format=plsc.PackFormat.INTERLEAVED)
                sb[e, pl.ds(m * 32, 16)] = lo * asp
                sb[e, pl.ds(m * 32 + 16, 16)] = hi * asp

    for half, z_hbm in enumerate((z0_hbm, z1_hbm)):
        plsc.subcore_barrier()
        pltpu.async_copy(z_hbm.at[dst_v.at[0]], rows0_v, semg0)
        pltpu.async_copy(z_hbm.at[dst_v.at[1]], rows1_v, semg1)

        def _pair(i, c):
            j0 = 2 * i
            pltpu.make_async_copy(z_hbm.at[dst_v.at[j0]], rows0_v,
                                  semg0).wait()

            @pl.when(i > 0)
            def _():
                pltpu.make_async_copy(sb0_v, acc_sh.at[src_v.at[j0]],
                                      sems0).wait()
            _scale(rows0_v, sb0_v, j0)

            @pl.when(j0 + 2 < nch)
            def _():
                pltpu.async_copy(z_hbm.at[dst_v.at[j0 + 2]], rows0_v, semg0)
            pltpu.async_copy(sb0_v, acc_sh.at[src_v.at[j0]], sems0,
                             add=True)

            j1 = 2 * i + 1
            pltpu.make_async_copy(z_hbm.at[dst_v.at[j1]], rows1_v,
                                  semg1).wait()

            @pl.when(i > 0)
            def _():
                pltpu.make_async_copy(sb1_v, acc_sh.at[src_v.at[j1]],
                                      sems1).wait()
            _scale(rows1_v, sb1_v, j1)

            @pl.when(j1 + 2 < nch)
            def _():
                pltpu.async_copy(z_hbm.at[dst_v.at[j1 + 2]], rows1_v, semg1)
            pltpu.async_copy(sb1_v, acc_sh.at[src_v.at[j1]], sems1,
                             add=True)
            return c
        lax.fori_loop(0, nch // 2, _pair, 0)
        pltpu.make_async_copy(sb0_v, acc_sh.at[src_v.at[0]], sems0).wait()
        pltpu.make_async_copy(sb1_v, acc_sh.at[src_v.at[0]], sems1).wait()

        plsc.subcore_barrier()
        pltpu.sync_copy(acc_sh.at[pl.ds(sid * SL, SL)],
                        outp_hbm.at[cid, half, pl.ds(sid * SL, SL)])
        if half == 0:
            lax.fori_loop(0, CH, _zr, 0)
            for kk in range(SL // CH):
                pltpu.sync_copy(
                    sb0_v, acc_sh.at[pl.ds(sid * SL + kk * CH, CH)])


def _k4_body(p_ref, o_ref):
    o_ref[:, :FH] = p_ref[0, 0] + p_ref[1, 0]
    o_ref[:, FH:] = p_ref[0, 1] + p_ref[1, 1]


def _k4(outp):
    return pl.pallas_call(
        _k4_body,
        grid=(10,),
        in_specs=[pl.BlockSpec((2, 2, 1000, FH), lambda i: (0, 0, i, 0))],
        out_specs=pl.BlockSpec((1000, N_FEAT), lambda i: (i, 0)),
        out_shape=jax.ShapeDtypeStruct((N_NODES, N_FEAT), jnp.float32),
    )(outp)


def kernel(x, edge_index, W_fc, a_w, a_b):
    ei = jnp.pad(edge_index.astype(jnp.int32),
                 ((0, 0), (0, E_PAD - N_EDGES))).reshape(2, NW, NCH, CH)
    perm = jnp.asarray(_PERM)
    w_p = W_fc[perm, :]
    a_pair = a_w.reshape(2, N_FEAT)[:, perm]
    bias = jnp.stack([a_b, jnp.zeros_like(a_b)]).reshape(2, 1)
    x_pad = jnp.pad(x, ((0, NPAD - N_NODES), (0, 0)))
    z0, z1, s_pair = _k1(x_pad, w_p, a_pair, bias)
    h, hsum_p = _k2(ei, s_pair)
    alpha, outp = _k3(ei, h, hsum_p, z0, z1)
    out = _k4(outp)
    return out, alpha.reshape(E_PAD)[:N_EDGES]
